# baseline (speedup 1.0000x reference)
import functools

import jax
import jax.numpy as jnp
from jax import lax
from jax.experimental import pallas as pl
from jax.experimental.pallas import tpu as pltpu
from jax.experimental.pallas import tpu_sc as plsc

SEQ = 50
BATCH = 4096
EMBED = 64
N_ROWS = SEQ * BATCH

NUM_CORES = 2
NUM_SUBCORES = 16
NW = NUM_CORES * NUM_SUBCORES
B_PER_W = N_ROWS // NW
CHUNK = 128
N_CHUNKS = B_PER_W // CHUNK
NBUF = 5
ROUNDS = N_CHUNKS // NBUF

_mesh = plsc.VectorSubcoreMesh(core_axis_name="c", subcore_axis_name="s")


def _emb_body(idx_hbm, table_hbm, out_hbm, idx_v, bufs, *sems):
    g_sems = sems[:NBUF]
    s_sems = sems[NBUF:]
    wid = lax.axis_index("s") * NUM_CORES + lax.axis_index("c")
    base = wid * B_PER_W

    pltpu.sync_copy(idx_hbm.at[wid], idx_v)

    def gather(j, b):
        return pltpu.make_async_copy(
            table_hbm.at[idx_v.at[j]], bufs.at[b], g_sems[b])

    def store(j, b):
        return pltpu.make_async_copy(
            bufs.at[b], out_hbm.at[pl.ds(base + j * CHUNK, CHUNK)], s_sems[b])

    for b in range(NBUF):
        gather(b, b).start()

    def round_body(G, _):
        for b in range(NBUF):
            j = G * NBUF + b
            gather(j, b).wait()
            st = store(j, b)
            st.start()

            @pl.when(G < ROUNDS - 1)
            def _():
                st.wait()
                gather(j + NBUF, b).start()

        return ()

    lax.fori_loop(0, ROUNDS, round_body, (), unroll=False)

    for b in range(NBUF):
        store((ROUNDS - 1) * NBUF + b, b).wait()


@functools.partial(jax.jit, donate_argnums=())
def _embedding_sc(idx, table):
    call = pl.kernel(
        _emb_body,
        out_type=jax.ShapeDtypeStruct((N_ROWS, EMBED), jnp.float32),
        mesh=_mesh,
        scratch_types=[
            pltpu.VMEM((N_CHUNKS, CHUNK), jnp.int32),
            pltpu.VMEM((NBUF, CHUNK, EMBED), jnp.float32),
        ] + [pltpu.SemaphoreType.DMA] * (2 * NBUF),
        compiler_params=pltpu.CompilerParams(use_tc_tiling_on_sc=False),
    )
    return call(idx, table)


def kernel(input, table):
    idx = input.reshape(NW, N_CHUNKS, CHUNK)
    out = _embedding_sc(idx, table)
    return out.reshape(-1, BATCH, EMBED)


# speedup vs baseline: 1.0577x; 1.0577x over previous
import functools

import jax
import jax.numpy as jnp
from jax import lax
from jax.experimental import pallas as pl
from jax.experimental.pallas import tpu as pltpu
from jax.experimental.pallas import tpu_sc as plsc

SEQ = 50
BATCH = 4096
EMBED = 64
PADDED = 128
N_ROWS = SEQ * BATCH

NUM_CORES = 2
NUM_SUBCORES = 16
NW = NUM_CORES * NUM_SUBCORES
B_PER_W = N_ROWS // NW
CHUNK = 128
N_CHUNKS = B_PER_W // CHUNK
NBUF = 5
ROUNDS = N_CHUNKS // NBUF

_mesh = plsc.VectorSubcoreMesh(core_axis_name="c", subcore_axis_name="s")


def _emb_body(idx_hbm, table_hbm, out_hbm, idx_v, bufs, *sems):
    g_sems = sems[:NBUF]
    s_sems = sems[NBUF:]
    wid = lax.axis_index("s") * NUM_CORES + lax.axis_index("c")
    base = wid * B_PER_W

    pltpu.sync_copy(idx_hbm.at[wid], idx_v)

    def gather(j, b):
        return pltpu.make_async_copy(
            table_hbm.at[idx_v.at[j]], bufs.at[b], g_sems[b])

    def store(j, b):
        return pltpu.make_async_copy(
            bufs.at[b, :, pl.ds(0, EMBED)],
            out_hbm.at[pl.ds(base + j * CHUNK, CHUNK)],
            s_sems[b],
        )

    for b in range(NBUF):
        gather(b, b).start()

    def round_body(G, _):
        for b in range(NBUF):
            j = G * NBUF + b
            gather(j, b).wait()
            st = store(j, b)
            st.start()

            @pl.when(G < ROUNDS - 1)
            def _():
                st.wait()
                gather(j + NBUF, b).start()

        return ()

    lax.fori_loop(0, ROUNDS, round_body, (), unroll=False)

    for b in range(NBUF):
        store((ROUNDS - 1) * NBUF + b, b).wait()


@jax.jit
def _embedding_sc(idx, padded_table):
    call = pl.kernel(
        _emb_body,
        out_type=jax.ShapeDtypeStruct((N_ROWS, EMBED), jnp.float32),
        mesh=_mesh,
        scratch_types=[
            pltpu.VMEM((N_CHUNKS, CHUNK), jnp.int32),
            pltpu.VMEM((NBUF, CHUNK, PADDED), jnp.float32),
        ] + [pltpu.SemaphoreType.DMA] * (2 * NBUF),
        compiler_params=pltpu.CompilerParams(use_tc_tiling_on_sc=False),
    )
    return call(idx, padded_table)


def kernel(input, table):
    padded = jnp.pad(table, ((0, 0), (0, PADDED - EMBED)))
    idx = input.reshape(NW, N_CHUNKS, CHUNK)
    out = _embedding_sc(idx, padded)
    return out.reshape(-1, BATCH, EMBED)


# speedup vs baseline: 1.1693x; 1.1055x over previous
import functools

import jax
import jax.numpy as jnp
from jax import lax
from jax.experimental import pallas as pl
from jax.experimental.pallas import tpu as pltpu
from jax.experimental.pallas import tpu_sc as plsc

SEQ = 50
BATCH = 4096
EMBED = 64
PADDED = 128
VOCAB_ROWS = 1000000
N_ROWS = SEQ * BATCH

NUM_CORES = 2
NUM_SUBCORES = 16
NW = NUM_CORES * NUM_SUBCORES
B_PER_W = N_ROWS // NW
CHUNK = 128
N_CHUNKS = B_PER_W // CHUNK
NBUF = 5
ROUNDS = N_CHUNKS // NBUF

TBLOCK = 2048
TGRID = -(-VOCAB_ROWS // TBLOCK)

_mesh = plsc.VectorSubcoreMesh(core_axis_name="c", subcore_axis_name="s")


def _transpose_body(tt_ref, out_ref):
    out_ref[:, 0:EMBED] = tt_ref[...].T


def _pad_transpose_tc(tt):
    return pl.pallas_call(
        _transpose_body,
        grid=(TGRID,),
        in_specs=[pl.BlockSpec((EMBED, TBLOCK), lambda i: (0, i))],
        out_specs=pl.BlockSpec((TBLOCK, PADDED), lambda i: (i, 0)),
        out_shape=jax.ShapeDtypeStruct((VOCAB_ROWS, PADDED), jnp.float32),
    )(tt)


def _emb_body(idx_hbm, table_hbm, out_hbm, idx_v, bufs, *sems):
    g_sems = sems[:NBUF]
    s_sems = sems[NBUF:]
    wid = lax.axis_index("s") * NUM_CORES + lax.axis_index("c")
    base = wid * B_PER_W

    pltpu.sync_copy(idx_hbm.at[wid], idx_v)

    def gather(j, b):
        return pltpu.make_async_copy(
            table_hbm.at[idx_v.at[j]], bufs.at[b], g_sems[b])

    def store(j, b):
        return pltpu.make_async_copy(
            bufs.at[b, :, pl.ds(0, EMBED)],
            out_hbm.at[pl.ds(base + j * CHUNK, CHUNK)],
            s_sems[b],
        )

    for b in range(NBUF):
        gather(b, b).start()

    def round_body(G, _):
        for b in range(NBUF):
            j = G * NBUF + b
            gather(j, b).wait()
            st = store(j, b)
            st.start()

            @pl.when(G < ROUNDS - 1)
            def _():
                st.wait()
                gather(j + NBUF, b).start()

        return ()

    lax.fori_loop(0, ROUNDS, round_body, (), unroll=False)

    for b in range(NBUF):
        store((ROUNDS - 1) * NBUF + b, b).wait()


@jax.jit
def _embedding_sc(idx, staged_table):
    call = pl.kernel(
        _emb_body,
        out_type=jax.ShapeDtypeStruct((N_ROWS, EMBED), jnp.float32),
        mesh=_mesh,
        scratch_types=[
            pltpu.VMEM((N_CHUNKS, CHUNK), jnp.int32),
            pltpu.VMEM((NBUF, CHUNK, PADDED), jnp.float32),
        ] + [pltpu.SemaphoreType.DMA] * (2 * NBUF),
        compiler_params=pltpu.CompilerParams(use_tc_tiling_on_sc=False),
    )
    return call(idx, staged_table)


def kernel(input, table):
    staged = _pad_transpose_tc(table.T)
    idx = input.reshape(NW, N_CHUNKS, CHUNK)
    out = _embedding_sc(idx, staged)
    return out.reshape(-1, BATCH, EMBED)


# speedup vs baseline: 1.6642x; 1.4233x over previous
import functools

import jax
import jax.numpy as jnp
from jax import lax
from jax.experimental import pallas as pl
from jax.experimental.pallas import tpu as pltpu
from jax.experimental.pallas import tpu_sc as plsc

SEQ = 50
BATCH = 4096
EMBED = 64
PADDED = 128
VOCAB_ROWS = 1000000
N_ROWS = SEQ * BATCH

NUM_CORES = 2
NUM_SUBCORES = 16
NW = NUM_CORES * NUM_SUBCORES
B_PER_W = N_ROWS // NW
CHUNK = 128
N_CHUNKS = B_PER_W // CHUNK
NBUF = 5
ROUNDS = N_CHUNKS // NBUF

TBLOCK = 8192
TGRID = -(-VOCAB_ROWS // TBLOCK)

_mesh = plsc.VectorSubcoreMesh(core_axis_name="c", subcore_axis_name="s")


def _transpose_body(tt_ref, out_ref):
    out_ref[:, 0:EMBED] = tt_ref[...].T


def _pad_transpose_tc(tt):
    return pl.pallas_call(
        _transpose_body,
        grid=(TGRID,),
        in_specs=[pl.BlockSpec((EMBED, TBLOCK), lambda i: (0, i))],
        out_specs=pl.BlockSpec((TBLOCK, PADDED), lambda i: (i, 0)),
        out_shape=jax.ShapeDtypeStruct((VOCAB_ROWS, PADDED), jnp.float32),
    )(tt)


def _emb_body(idx_hbm, table_hbm, out_hbm, idx_v, bufs, *sems):
    g_sems = sems[:NBUF]
    s_sems = sems[NBUF:]
    wid = lax.axis_index("s") * NUM_CORES + lax.axis_index("c")
    base = wid * B_PER_W

    pltpu.sync_copy(idx_hbm.at[wid], idx_v)

    def gather(j, b):
        return pltpu.make_async_copy(
            table_hbm.at[idx_v.at[j]], bufs.at[b], g_sems[b])

    def store(j, b):
        return pltpu.make_async_copy(
            bufs.at[b, :, pl.ds(0, EMBED)],
            out_hbm.at[pl.ds(base + j * CHUNK, CHUNK)],
            s_sems[b],
        )

    for b in range(NBUF):
        gather(b, b).start()

    def round_body(G, _):
        for b in range(NBUF):
            j = G * NBUF + b
            gather(j, b).wait()
            st = store(j, b)
            st.start()

            @pl.when(G < ROUNDS - 1)
            def _():
                st.wait()
                gather(j + NBUF, b).start()

        return ()

    lax.fori_loop(0, ROUNDS, round_body, (), unroll=False)

    for b in range(NBUF):
        store((ROUNDS - 1) * NBUF + b, b).wait()


@jax.jit
def _embedding_sc(idx, staged_table):
    call = pl.kernel(
        _emb_body,
        out_type=jax.ShapeDtypeStruct((N_ROWS, EMBED), jnp.float32),
        mesh=_mesh,
        scratch_types=[
            pltpu.VMEM((N_CHUNKS, CHUNK), jnp.int32),
            pltpu.VMEM((NBUF, CHUNK, PADDED), jnp.float32),
        ] + [pltpu.SemaphoreType.DMA] * (2 * NBUF),
        compiler_params=pltpu.CompilerParams(use_tc_tiling_on_sc=False),
    )
    return call(idx, staged_table)


def kernel(input, table):
    staged = _pad_transpose_tc(table.T)
    idx = input.reshape(NW, N_CHUNKS, CHUNK)
    out = _embedding_sc(idx, staged)
    return out.reshape(-1, BATCH, EMBED)


# speedup vs baseline: 1.7462x; 1.0492x over previous
import functools

import jax
import jax.numpy as jnp
from jax import lax
from jax.experimental import pallas as pl
from jax.experimental.pallas import tpu as pltpu
from jax.experimental.pallas import tpu_sc as plsc

SEQ = 50
BATCH = 4096
EMBED = 64
PADDED = 128
VOCAB_ROWS = 1000000
N_ROWS = SEQ * BATCH

NUM_CORES = 2
NUM_SUBCORES = 16
NW = NUM_CORES * NUM_SUBCORES
B_PER_W = N_ROWS // NW
CHUNK = 128
N_CHUNKS = B_PER_W // CHUNK
NBUF = 5
ROUNDS = N_CHUNKS // NBUF

TBLOCK = 16384
TGRID = -(-VOCAB_ROWS // TBLOCK)

_mesh = plsc.VectorSubcoreMesh(core_axis_name="c", subcore_axis_name="s")


def _transpose_body(tt_ref, out_ref):
    out_ref[:, 0:EMBED] = tt_ref[...].T


def _pad_transpose_tc(tt):
    return pl.pallas_call(
        _transpose_body,
        grid=(TGRID,),
        in_specs=[pl.BlockSpec((EMBED, TBLOCK), lambda i: (0, i))],
        out_specs=pl.BlockSpec((TBLOCK, PADDED), lambda i: (i, 0)),
        out_shape=jax.ShapeDtypeStruct((VOCAB_ROWS, PADDED), jnp.float32),
    )(tt)


def _emb_body(idx_hbm, table_hbm, out_hbm, idx_v, bufs, *sems):
    g_sems = sems[:NBUF]
    s_sems = sems[NBUF:]
    wid = lax.axis_index("s") * NUM_CORES + lax.axis_index("c")
    base = wid * B_PER_W

    pltpu.sync_copy(idx_hbm.at[wid], idx_v)

    def gather(j, b):
        return pltpu.make_async_copy(
            table_hbm.at[idx_v.at[j]], bufs.at[b], g_sems[b])

    def store(j, b):
        return pltpu.make_async_copy(
            bufs.at[b, :, pl.ds(0, EMBED)],
            out_hbm.at[pl.ds(base + j * CHUNK, CHUNK)],
            s_sems[b],
        )

    for b in range(NBUF):
        gather(b, b).start()

    def round_body(G, _):
        for b in range(NBUF):
            j = G * NBUF + b
            gather(j, b).wait()
            st = store(j, b)
            st.start()

            @pl.when(G < ROUNDS - 1)
            def _():
                st.wait()
                gather(j + NBUF, b).start()

        return ()

    lax.fori_loop(0, ROUNDS, round_body, (), unroll=False)

    for b in range(NBUF):
        store((ROUNDS - 1) * NBUF + b, b).wait()


@jax.jit
def _embedding_sc(idx, staged_table):
    call = pl.kernel(
        _emb_body,
        out_type=jax.ShapeDtypeStruct((N_ROWS, EMBED), jnp.float32),
        mesh=_mesh,
        scratch_types=[
            pltpu.VMEM((N_CHUNKS, CHUNK), jnp.int32),
            pltpu.VMEM((NBUF, CHUNK, PADDED), jnp.float32),
        ] + [pltpu.SemaphoreType.DMA] * (2 * NBUF),
        compiler_params=pltpu.CompilerParams(use_tc_tiling_on_sc=False),
    )
    return call(idx, staged_table)


def kernel(input, table):
    staged = _pad_transpose_tc(table.T)
    idx = input.reshape(NW, N_CHUNKS, CHUNK)
    out = _embedding_sc(idx, staged)
    return out.reshape(-1, BATCH, EMBED)


# speedup vs baseline: 1.7729x; 1.0153x over previous
import functools

import jax
import jax.numpy as jnp
from jax import lax
from jax.experimental import pallas as pl
from jax.experimental.pallas import tpu as pltpu
from jax.experimental.pallas import tpu_sc as plsc

SEQ = 50
BATCH = 4096
EMBED = 64
PADDED = 128
VOCAB_ROWS = 1000000
N_ROWS = SEQ * BATCH

NUM_CORES = 2
NUM_SUBCORES = 16
NW = NUM_CORES * NUM_SUBCORES
B_PER_W = N_ROWS // NW
CHUNK = 128
N_CHUNKS = B_PER_W // CHUNK
NBUF = 5
ROUNDS = N_CHUNKS // NBUF

TBLOCK = 32768
TGRID = -(-VOCAB_ROWS // TBLOCK)

_mesh = plsc.VectorSubcoreMesh(core_axis_name="c", subcore_axis_name="s")


def _transpose_body(tt_ref, out_ref):
    out_ref[:, 0:EMBED] = tt_ref[...].T


def _pad_transpose_tc(tt):
    return pl.pallas_call(
        _transpose_body,
        grid=(TGRID,),
        in_specs=[pl.BlockSpec((EMBED, TBLOCK), lambda i: (0, i))],
        out_specs=pl.BlockSpec((TBLOCK, PADDED), lambda i: (i, 0)),
        out_shape=jax.ShapeDtypeStruct((VOCAB_ROWS, PADDED), jnp.float32),
    )(tt)


def _emb_body(idx_hbm, table_hbm, out_hbm, idx_v, bufs, *sems):
    g_sems = sems[:NBUF]
    s_sems = sems[NBUF:]
    wid = lax.axis_index("s") * NUM_CORES + lax.axis_index("c")
    base = wid * B_PER_W

    pltpu.sync_copy(idx_hbm.at[wid], idx_v)

    def gather(j, b):
        return pltpu.make_async_copy(
            table_hbm.at[idx_v.at[j]], bufs.at[b], g_sems[b])

    def store(j, b):
        return pltpu.make_async_copy(
            bufs.at[b, :, pl.ds(0, EMBED)],
            out_hbm.at[pl.ds(base + j * CHUNK, CHUNK)],
            s_sems[b],
        )

    for b in range(NBUF):
        gather(b, b).start()

    def round_body(G, _):
        for b in range(NBUF):
            j = G * NBUF + b
            gather(j, b).wait()
            st = store(j, b)
            st.start()

            @pl.when(G < ROUNDS - 1)
            def _():
                st.wait()
                gather(j + NBUF, b).start()

        return ()

    lax.fori_loop(0, ROUNDS, round_body, (), unroll=False)

    for b in range(NBUF):
        store((ROUNDS - 1) * NBUF + b, b).wait()


@jax.jit
def _embedding_sc(idx, staged_table):
    call = pl.kernel(
        _emb_body,
        out_type=jax.ShapeDtypeStruct((N_ROWS, EMBED), jnp.float32),
        mesh=_mesh,
        scratch_types=[
            pltpu.VMEM((N_CHUNKS, CHUNK), jnp.int32),
            pltpu.VMEM((NBUF, CHUNK, PADDED), jnp.float32),
        ] + [pltpu.SemaphoreType.DMA] * (2 * NBUF),
        compiler_params=pltpu.CompilerParams(use_tc_tiling_on_sc=False),
    )
    return call(idx, staged_table)


def kernel(input, table):
    staged = _pad_transpose_tc(table.T)
    idx = input.reshape(NW, N_CHUNKS, CHUNK)
    out = _embedding_sc(idx, staged)
    return out.reshape(-1, BATCH, EMBED)


# speedup vs baseline: 2.0634x; 1.1639x over previous
import functools

import jax
import jax.numpy as jnp
from jax import lax
from jax.experimental import pallas as pl
from jax.experimental.pallas import tpu as pltpu
from jax.experimental.pallas import tpu_sc as plsc

SEQ = 50
BATCH = 4096
EMBED = 64
PADDED = 128
VOCAB_ROWS = 1000000
N_ROWS = SEQ * BATCH

NUM_CORES = 2
NUM_SUBCORES = 16
NW = NUM_CORES * NUM_SUBCORES
B_PER_W = N_ROWS // NW
CHUNK = 128
N_CHUNKS = B_PER_W // CHUNK
NBUF = 5
ROUNDS = N_CHUNKS // NBUF

TBLOCK = 32768
TGRID = -(-VOCAB_ROWS // TBLOCK)

_mesh = plsc.VectorSubcoreMesh(core_axis_name="c", subcore_axis_name="s")


def _transpose_body(tt_ref, out_ref):
    out_ref[:, 0:EMBED] = tt_ref[...].T


def _pad_transpose_tc(tt):
    return pl.pallas_call(
        _transpose_body,
        grid=(TGRID,),
        in_specs=[pl.BlockSpec((EMBED, TBLOCK), lambda i: (0, i))],
        out_specs=pl.BlockSpec((TBLOCK, PADDED), lambda i: (i, 0)),
        out_shape=jax.ShapeDtypeStruct((VOCAB_ROWS, PADDED), jnp.float32),
    )(tt)


def _emb_body(idx_hbm, table_hbm, out_hbm, idx_v, bufs, *sems):
    g_sems = sems[:NBUF]
    s_sems = sems[NBUF:]
    wid = lax.axis_index("s") * NUM_CORES + lax.axis_index("c")
    base = wid * B_PER_W

    pltpu.sync_copy(idx_hbm.at[wid], idx_v)

    def gather(j, b):
        return pltpu.make_async_copy(
            table_hbm.at[idx_v.at[j]], bufs.at[b], g_sems[b])

    def store(j, b):
        return pltpu.make_async_copy(
            bufs.at[b],
            out_hbm.at[pl.ds(base + j * CHUNK, CHUNK)],
            s_sems[b],
        )

    for b in range(NBUF):
        gather(b, b).start()

    def round_body(G, _):
        for b in range(NBUF):
            j = G * NBUF + b
            gather(j, b).wait()
            st = store(j, b)
            st.start()

            @pl.when(G < ROUNDS - 1)
            def _():
                st.wait()
                gather(j + NBUF, b).start()

        return ()

    lax.fori_loop(0, ROUNDS, round_body, (), unroll=False)

    for b in range(NBUF):
        store((ROUNDS - 1) * NBUF + b, b).wait()


@jax.jit
def _embedding_sc(idx, staged_table):
    call = pl.kernel(
        _emb_body,
        out_type=jax.ShapeDtypeStruct((N_ROWS, PADDED), jnp.float32),
        mesh=_mesh,
        scratch_types=[
            pltpu.VMEM((N_CHUNKS, CHUNK), jnp.int32),
            pltpu.VMEM((NBUF, CHUNK, PADDED), jnp.float32),
        ] + [pltpu.SemaphoreType.DMA] * (2 * NBUF),
        compiler_params=pltpu.CompilerParams(use_tc_tiling_on_sc=False),
    )
    return call(idx, staged_table)


def kernel(input, table):
    staged = _pad_transpose_tc(table.T)
    idx = input.reshape(NW, N_CHUNKS, CHUNK)
    out = _embedding_sc(idx, staged)
    return out.reshape(-1, BATCH, PADDED)[:, :, :EMBED]


# speedup vs baseline: 2.1564x; 1.0450x over previous
import functools

import jax
import jax.numpy as jnp
from jax import lax
from jax.experimental import pallas as pl
from jax.experimental.pallas import tpu as pltpu
from jax.experimental.pallas import tpu_sc as plsc

SEQ = 50
BATCH = 4096
EMBED = 64
PADDED = 128
VOCAB_ROWS = 1000000
N_ROWS = SEQ * BATCH

NUM_CORES = 2
NUM_SUBCORES = 16
NW = NUM_CORES * NUM_SUBCORES
B_PER_W = N_ROWS // NW
CHUNK = 128
N_CHUNKS = B_PER_W // CHUNK
NBUF = 5
ROUNDS = N_CHUNKS // NBUF

TBLOCK = 32768
TGRID = -(-VOCAB_ROWS // TBLOCK)

_mesh = plsc.VectorSubcoreMesh(core_axis_name="c", subcore_axis_name="s")


def _transpose_body(tt_ref, out_ref):
    out_ref[:, 0:EMBED] = tt_ref[...].T


def _pad_transpose_tc(tt):
    return pl.pallas_call(
        _transpose_body,
        grid=(TGRID,),
        in_specs=[pl.BlockSpec((EMBED, TBLOCK), lambda i: (0, i))],
        out_specs=pl.BlockSpec((TBLOCK, PADDED), lambda i: (i, 0)),
        out_shape=jax.ShapeDtypeStruct((VOCAB_ROWS, PADDED), jnp.float32),
    )(tt)


def _emb_body(idx_hbm, table_hbm, out_hbm, idx_v, bufs, *sems):
    g_sems = sems[:NBUF]
    s_sems = sems[NBUF:]
    wid = lax.axis_index("s") * NUM_CORES + lax.axis_index("c")
    base = wid * B_PER_W

    pltpu.sync_copy(idx_hbm.at[wid], idx_v)

    def gather(j, b):
        return pltpu.make_async_copy(
            table_hbm.at[idx_v.at[j]], bufs.at[b], g_sems[b])

    def store(j, b):
        return pltpu.make_async_copy(
            bufs.at[b, :, pl.ds(0, EMBED)],
            out_hbm.at[pl.ds(base + j * CHUNK, CHUNK), pl.ds(0, EMBED)],
            s_sems[b],
        )

    for b in range(NBUF):
        gather(b, b).start()

    def round_body(G, _):
        for b in range(NBUF):
            j = G * NBUF + b
            gather(j, b).wait()
            st = store(j, b)
            st.start()

            @pl.when(G < ROUNDS - 1)
            def _():
                st.wait()
                gather(j + NBUF, b).start()

        return ()

    lax.fori_loop(0, ROUNDS, round_body, (), unroll=False)

    for b in range(NBUF):
        store((ROUNDS - 1) * NBUF + b, b).wait()


@jax.jit
def _embedding_sc(idx, staged_table):
    call = pl.kernel(
        _emb_body,
        out_type=jax.ShapeDtypeStruct((N_ROWS, PADDED), jnp.float32),
        mesh=_mesh,
        scratch_types=[
            pltpu.VMEM((N_CHUNKS, CHUNK), jnp.int32),
            pltpu.VMEM((NBUF, CHUNK, PADDED), jnp.float32),
        ] + [pltpu.SemaphoreType.DMA] * (2 * NBUF),
        compiler_params=pltpu.CompilerParams(use_tc_tiling_on_sc=False),
    )
    return call(idx, staged_table)


def kernel(input, table):
    staged = _pad_transpose_tc(table.T)
    idx = input.reshape(NW, N_CHUNKS, CHUNK)
    out = _embedding_sc(idx, staged)
    return out.reshape(-1, BATCH, PADDED)[:, :, :EMBED]
